# baseline (speedup 1.0000x reference)
import jax
import jax.numpy as jnp
from jax import lax
from jax.experimental import pallas as pl
from jax.experimental.pallas import tpu as pltpu

V = 100000
D = 32
C = 50
H = 300
BV = 4096
NB = (V + BV - 1) // BV


def _body(idx_ref, table_ref, w1_ref, b1_ref, w2_ref, b2_ref, out_ref,
          emb_ref, logits_ref, h_ref, m_ref, s_ref, sem):
    j = pl.program_id(0)

    @pl.when(j == 0)
    def _init():
        cps = [
            pltpu.make_async_copy(
                table_ref.at[pl.ds(idx_ref[c], 1)],
                emb_ref.at[pl.ds(c, 1)],
                sem,
            )
            for c in range(C)
        ]
        for cp in cps:
            cp.start()
        for cp in cps:
            cp.wait()
        acc = b1_ref[...]
        for c in range(C):
            acc = acc + jnp.dot(emb_ref[pl.ds(c, 1), :],
                                w1_ref[pl.ds(D * c, D), :],
                                preferred_element_type=jnp.float32)
        h_ref[...] = jnp.maximum(acc, 0.0)
        m_ref[...] = jnp.full((1, 1), -jnp.inf, jnp.float32)
        s_ref[...] = jnp.zeros((1, 1), jnp.float32)

    def _merge(bm, se):
        m_old = m_ref[...]
        m_new = jnp.maximum(m_old, bm)
        s_ref[...] = (s_ref[...] * jnp.exp(m_old - m_new)
                      + se * jnp.exp(bm - m_new))
        m_ref[...] = m_new

    def _stats(lg):
        bm = jnp.max(lg, axis=1, keepdims=True)
        se = jnp.sum(jnp.exp(lg - bm), axis=1, keepdims=True)
        return bm, se

    logits = jnp.dot(h_ref[...].astype(jnp.bfloat16),
                     w2_ref[...].astype(jnp.bfloat16),
                     preferred_element_type=jnp.float32) + b2_ref[...]
    logits_ref[j] = logits

    @pl.when(j > 0)
    def _prev_stats():
        bm, se = _stats(logits_ref[j - 1])
        _merge(bm, se)

    @pl.when(j == NB - 1)
    def _fin():
        VW = V - (NB - 1) * BV
        bm, se = _stats(logits_ref[NB - 1][:, :VW])
        _merge(bm, se)
        lse = m_ref[...] + jnp.log(s_ref[...])
        for j2 in range(NB):
            width = min(BV, V - j2 * BV)
            out_ref[:, pl.ds(j2 * BV, width)] = (
                logits_ref[j2][:, :width] - lse)


def _call(idx, table, W1, b1, W2, b2):
    grid_spec = pltpu.PrefetchScalarGridSpec(
        num_scalar_prefetch=1,
        grid=(NB,),
        in_specs=[
            pl.BlockSpec(memory_space=pl.ANY),
            pl.BlockSpec((C * D, H), lambda j, idx: (0, 0)),
            pl.BlockSpec((1, H), lambda j, idx: (0, 0)),
            pl.BlockSpec((H, BV), lambda j, idx: (0, j)),
            pl.BlockSpec((1, BV), lambda j, idx: (0, j)),
        ],
        out_specs=pl.BlockSpec((1, V), lambda j, idx: (0, 0)),
        scratch_shapes=[
            pltpu.VMEM((C, D), jnp.float32),
            pltpu.VMEM((NB, 1, BV), jnp.float32),
            pltpu.VMEM((1, H), jnp.float32),
            pltpu.VMEM((1, 1), jnp.float32),
            pltpu.VMEM((1, 1), jnp.float32),
            pltpu.SemaphoreType.DMA,
        ],
    )
    return pl.pallas_call(
        _body,
        grid_spec=grid_spec,
        out_shape=jax.ShapeDtypeStruct((1, V), jnp.float32),
    )(idx, table, W1, b1, W2, b2)


def kernel(inp, table, W1, b1, W2, b2):
    return _call(inp.astype(jnp.int32), table, W1,
                 b1.reshape(1, H), W2, b2.reshape(1, V))


# speedup vs baseline: 1.0465x; 1.0465x over previous
import jax
import jax.numpy as jnp
from jax import lax
from jax.experimental import pallas as pl
from jax.experimental.pallas import tpu as pltpu

V = 100000
D = 32
C = 50
H = 300
BV = 8192
NB = (V + BV - 1) // BV
NBUF = 3
VW = V - (NB - 1) * BV


def _w2_copy(w2_hbm, bufs, sems, block, slot):
    return pltpu.make_async_copy(
        w2_hbm.at[:, pl.ds(block * BV, BV)],
        bufs.at[slot],
        sems.at[slot],
    )


def _body(idx_ref, table_ref, w1_ref, b1_ref, w2_hbm, w2_tail_ref, b2_ref,
          out_ref, bufs, emb_ref, logits_ref, h_ref, m_ref, s_ref, sems,
          gsem):
    j = pl.program_id(0)

    @pl.when(j == 0)
    def _init():
        for b in range(min(NBUF - 1, NB - 1)):
            _w2_copy(w2_hbm, bufs, sems, b, b).start()
        cps = [
            pltpu.make_async_copy(
                table_ref.at[pl.ds(idx_ref[c], 1)],
                emb_ref.at[pl.ds(c, 1)],
                gsem,
            )
            for c in range(C)
        ]
        for cp in cps:
            cp.start()
        for cp in cps:
            cp.wait()
        acc = b1_ref[...]
        for c in range(C):
            acc = acc + jnp.dot(emb_ref[pl.ds(c, 1), :],
                                w1_ref[pl.ds(D * c, D), :],
                                preferred_element_type=jnp.float32)
        h_ref[...] = jnp.maximum(acc, 0.0)
        m_ref[...] = jnp.full((1, 1), -jnp.inf, jnp.float32)
        s_ref[...] = jnp.zeros((1, 1), jnp.float32)

    nxt = j + NBUF - 1

    @pl.when(nxt < NB - 1)
    def _prefetch():
        _w2_copy(w2_hbm, bufs, sems, nxt, lax.rem(nxt, NBUF)).start()

    slot = lax.rem(j, NBUF)

    def _merge(bm, se):
        m_old = m_ref[...]
        m_new = jnp.maximum(m_old, bm)
        s_ref[...] = (s_ref[...] * jnp.exp(m_old - m_new)
                      + se * jnp.exp(bm - m_new))
        m_ref[...] = m_new

    def _stats(lg):
        bm = jnp.max(lg, axis=1, keepdims=True)
        se = jnp.sum(jnp.exp(lg - bm), axis=1, keepdims=True)
        return bm, se

    @pl.when(j < NB - 1)
    def _main_matmul():
        _w2_copy(w2_hbm, bufs, sems, j, slot).wait()
        logits_ref[j] = jnp.dot(
            h_ref[...].astype(jnp.bfloat16),
            bufs[slot].astype(jnp.bfloat16),
            preferred_element_type=jnp.float32) + b2_ref[...]

    @pl.when(j == NB - 1)
    def _tail_matmul():
        logits_ref[j] = jnp.dot(
            h_ref[...].astype(jnp.bfloat16),
            w2_tail_ref[...].astype(jnp.bfloat16),
            preferred_element_type=jnp.float32) + b2_ref[...]

    @pl.when(j > 0)
    def _prev_stats():
        bm, se = _stats(logits_ref[j - 1])
        _merge(bm, se)

    @pl.when(j == NB - 1)
    def _fin():
        VW = V - (NB - 1) * BV
        bm, se = _stats(logits_ref[NB - 1][:, :VW])
        _merge(bm, se)
        lse = m_ref[...] + jnp.log(s_ref[...])
        for j2 in range(NB):
            width = min(BV, V - j2 * BV)
            out_ref[:, pl.ds(j2 * BV, width)] = (
                logits_ref[j2][:, :width] - lse)


def _call(idx, table, W1, b1, W2, b2):
    grid_spec = pltpu.PrefetchScalarGridSpec(
        num_scalar_prefetch=1,
        grid=(NB,),
        in_specs=[
            pl.BlockSpec(memory_space=pl.ANY),
            pl.BlockSpec((C * D, H), lambda j, idx: (0, 0)),
            pl.BlockSpec((1, H), lambda j, idx: (0, 0)),
            pl.BlockSpec(memory_space=pl.ANY),
            pl.BlockSpec((H, BV), lambda j, idx: (0, NB - 1)),
            pl.BlockSpec((1, BV), lambda j, idx: (0, j)),
        ],
        out_specs=pl.BlockSpec((1, V), lambda j, idx: (0, 0)),
        scratch_shapes=[
            pltpu.VMEM((NBUF, H, BV), jnp.float32),
            pltpu.VMEM((C, D), jnp.float32),
            pltpu.VMEM((NB, 1, BV), jnp.float32),
            pltpu.VMEM((1, H), jnp.float32),
            pltpu.VMEM((1, 1), jnp.float32),
            pltpu.VMEM((1, 1), jnp.float32),
            pltpu.SemaphoreType.DMA((NBUF,)),
            pltpu.SemaphoreType.DMA,
        ],
    )
    return pl.pallas_call(
        _body,
        grid_spec=grid_spec,
        out_shape=jax.ShapeDtypeStruct((1, V), jnp.float32),
    )(idx, table, W1, b1, W2, W2, b2)


def kernel(inp, table, W1, b1, W2, b2):
    return _call(inp.astype(jnp.int32), table, W1,
                 b1.reshape(1, H), W2, b2.reshape(1, V))
